# baseline (speedup 1.0000x reference)
import functools

import jax
import jax.numpy as jnp
from jax.experimental import pallas as pl
from jax.experimental.pallas import tpu as pltpu

EMBED = 32
LINEAR = 32
LSTM = 32
MSG = 32
N_STEPS = 4
P = 81
DEG = 20
E = P * DEG
BL = 4
S = BL * LINEAR
NSUB = 4
M = NSUB * P
EE = NSUB * E


def _rrn_block(*refs):
    (xt_ref, gs_ref, st_ref, r_ref, embk_ref, cellb_ref,
     prek_ref, preb_ref,
     w0k_ref, msgb0_ref, msgk_ref, msgb_ref,
     combk_ref, postxk_ref, postb0_ref, postk_ref, postb_ref,
     wihk_ref, whhk_ref, lstmb_ref,
     outk_ref, outb_ref, out_ref) = refs
    f32 = jnp.float32
    bf16 = jnp.bfloat16
    dot = functools.partial(jnp.dot, preferred_element_type=f32)
    dbb = functools.partial(jnp.dot, preferred_element_type=bf16)
    dnf = lambda u, w: jnp.dot(u.astype(bf16), w, preferred_element_type=f32)

    gs = gs_ref[...]
    st = st_ref[...]
    rmat = r_ref[...]
    preb = preb_ref[...]
    msgb = msgb_ref[...]
    postb = postb_ref[...]
    lstmb = lstmb_ref[...]
    cellb = cellb_ref[...]
    w0k = w0k_ref[...]
    msgb0 = msgb0_ref[...]

    xb = xt_ref[0]
    xrep = dot(xb, rmat)
    vals = (jax.lax.broadcasted_iota(jnp.int32, (M, BL * 10), 1) % 10).astype(f32)
    oh = (xrep == vals).astype(f32)
    z = jax.nn.relu(dot(oh, embk_ref[...]) + cellb)
    z = jax.nn.relu(dot(z, prek_ref[0]) + preb[0][None])
    z = jax.nn.relu(dot(z, prek_ref[1]) + preb[1][None])
    x0 = dot(z, prek_ref[2]) + preb[2][None]

    xpost = dot(x0, postxk_ref[...]) + postb0_ref[...]
    cur = x0
    hs = jnp.zeros((M, S), f32)
    cs = jnp.zeros((M, S), f32)

    for t in range(N_STEPS):
        curb = cur.astype(bf16)
        ab = jnp.dot(curb, w0k, preferred_element_type=f32)
        a = ab[:, :S].astype(bf16)
        b = ab[:, S:] + msgb0
        ae = jnp.dot(gs, a, preferred_element_type=f32)
        be = jnp.broadcast_to(b[:, None, :], (M, DEG, S)).reshape(EE, S)
        e = jax.nn.relu(ae + be)
        e = jax.nn.relu(dnf(e, msgk_ref[0]) + msgb[0][None])
        e = jax.nn.relu(dnf(e, msgk_ref[1]) + msgb[1][None])
        aggs = jnp.dot(st, e.astype(bf16), preferred_element_type=f32)

        z = jax.nn.relu(dot(aggs, combk_ref[...]) + xpost)
        z = jax.nn.relu(dot(z, postk_ref[0]) + postb[0][None])
        z = jax.nn.relu(dot(z, postk_ref[1]) + postb[1][None])

        gates = dot(z, wihk_ref[...]) + lstmb
        if t > 0:
            gates = gates + dot(hs, whhk_ref[...])
        ig = jax.nn.sigmoid(gates[:, 0 * S:1 * S])
        gg = jnp.tanh(gates[:, 2 * S:3 * S])
        og = jax.nn.sigmoid(gates[:, 3 * S:4 * S])
        if t > 0:
            fg = jax.nn.sigmoid(gates[:, 1 * S:2 * S])
            cs = fg * cs + ig * gg
        else:
            cs = ig * gg
        hs = og * jnp.tanh(cs)
        cur = cs

    out_ref[0] = dot(cur, outk_ref[...]) + outb_ref[...]


def kernel(x, edge_index, rows_emb, cols_emb, init_emb, pre_W0, pre_b0,
           pre_Wr, pre_br, msg_W0, msg_b0, msg_Wr, msg_br, post_W0, post_b0,
           post_Wr, post_br, lstm_Wih, lstm_Whh, lstm_bih, lstm_bhh,
           out_W, out_b):
    f32 = jnp.float32
    bf16 = jnp.bfloat16
    batch = x.shape[0]
    G = batch // (BL * NSUB)
    eye = jnp.eye(BL, dtype=f32)

    def kron(w):
        k, f = w.shape
        return jnp.einsum('ab,kf->akbf', eye, w).reshape(BL * k, BL * f)

    def tileb(bvec):
        return jnp.tile(bvec, (BL,)).reshape(1, -1)

    e0 = edge_index[:E]
    order = jnp.argsort(e0[:, 1], stable=True)
    gs1 = jax.nn.one_hot(e0[order, 0], P, dtype=f32)
    gs = jnp.einsum('ab,ef->aebf', jnp.eye(NSUB, dtype=f32),
                    gs1).reshape(EE, M).astype(bf16)
    st = jnp.repeat(jnp.eye(M, dtype=bf16), DEG, axis=1)

    node = jnp.arange(P)
    cellb = (jnp.take(rows_emb, node // 9, axis=0) @ pre_W0[EMBED:2 * EMBED]
             + jnp.take(cols_emb, node % 9, axis=0) @ pre_W0[2 * EMBED:]
             + pre_b0)
    cellb_t = jnp.tile(cellb, (NSUB, BL))
    emb_proj = init_emb @ pre_W0[:EMBED]
    embk = jnp.einsum('ab,vf->avbf', eye, emb_proj).reshape(BL * 10, S)
    rmat = jnp.repeat(eye, 10, axis=1)

    prek = jnp.stack([kron(pre_Wr[i]) for i in range(3)])
    preb = jnp.stack([tileb(pre_br[i])[0] for i in range(3)])
    w0k = jnp.concatenate([kron(msg_W0[:LINEAR]), kron(msg_W0[LINEAR:])],
                          axis=1).astype(bf16)
    msgb0 = tileb(msg_b0)
    msgk = jnp.stack([kron(msg_Wr[i]) for i in range(2)]).astype(bf16)
    msgb = jnp.stack([tileb(msg_br[i])[0] for i in range(2)])

    combk = kron(msg_Wr[2] @ post_W0[:MSG])
    postxk = kron(post_W0[MSG:])
    postb0 = tileb(post_b0 + DEG * (msg_br[2] @ post_W0[:MSG]))
    postk = jnp.stack([kron(post_Wr[i]) for i in range(2)])
    postb = jnp.stack([tileb(post_br[i])[0] for i in range(2)])

    wih_f = post_Wr[2] @ lstm_Wih
    w4 = wih_f.reshape(LINEAR, 4, LSTM)
    wihk = jnp.einsum('ab,kgf->akgbf', eye, w4).reshape(S, 4 * S)
    h4 = lstm_Whh.reshape(LSTM, 4, LSTM)
    whhk = jnp.einsum('ab,kgf->akgbf', eye, h4).reshape(S, 4 * S)
    lb = lstm_bih + lstm_bhh + post_br[2] @ lstm_Wih
    lb4 = lb.reshape(4, 1, LSTM)
    lstmb = jnp.broadcast_to(lb4, (4, BL, LSTM)).reshape(1, 4 * S)

    outk = kron(out_W)
    outb = tileb(out_b)

    xt = (x.reshape(G, NSUB, BL, P).transpose(0, 1, 3, 2)
          .reshape(G, M, BL).astype(f32))

    full = lambda arr: pl.BlockSpec(arr.shape, lambda i: (0,) * arr.ndim)
    weights = [gs, st, rmat, embk, cellb_t, prek, preb, w0k, msgb0,
               msgk, msgb, combk, postxk, postb0, postk, postb,
               wihk, whhk, lstmb, outk, outb]
    in_specs = [pl.BlockSpec((1, M, BL), lambda i: (i, 0, 0))]
    in_specs += [full(w) for w in weights]

    out = pl.pallas_call(
        _rrn_block,
        grid=(G,),
        in_specs=in_specs,
        out_specs=pl.BlockSpec((1, M, BL * 9), lambda i: (i, 0, 0)),
        out_shape=jax.ShapeDtypeStruct((G, M, BL * 9), f32),
        compiler_params=pltpu.CompilerParams(
            dimension_semantics=("arbitrary",)),
    )(xt, *weights)

    return (out.reshape(G, NSUB, P, BL, 9).transpose(0, 1, 3, 2, 4)
            .reshape(batch, P, 9))


# speedup vs baseline: 2.2370x; 2.2370x over previous
import functools

import jax
import jax.numpy as jnp
from jax.experimental import pallas as pl
from jax.experimental.pallas import tpu as pltpu

EMBED = 32
LINEAR = 32
LSTM = 32
MSG = 32
N_STEPS = 4
P = 81
DEG = 20
E = P * DEG
BL = 8
S = BL * LINEAR
NSUB = 2
M = NSUB * P
EE = NSUB * E


def _rrn_block(*refs):
    (xt_ref, gs_ref, st_ref, r_ref, embk_ref, cellb_ref,
     prek_ref, preb_ref,
     w0k_ref, msgb0_ref, msgk_ref, msgb_ref,
     combk_ref, postxk_ref, postb0_ref, postk_ref, postb_ref,
     wihk_ref, whhk_ref, lstmb_ref,
     outk_ref, outb_ref, out_ref) = refs
    f32 = jnp.float32
    bf16 = jnp.bfloat16
    dot = functools.partial(jnp.dot, preferred_element_type=f32)
    dbb = functools.partial(jnp.dot, preferred_element_type=bf16)
    dnf = lambda u, w: jnp.dot(u.astype(bf16), w, preferred_element_type=f32)

    gs = gs_ref[...]
    st = st_ref[...]
    rmat = r_ref[...]
    preb = preb_ref[...]
    msgb = msgb_ref[...]
    postb = postb_ref[...]
    lstmb = lstmb_ref[...]
    cellb = cellb_ref[...]
    w0k = w0k_ref[...]
    msgb0 = msgb0_ref[...]

    xb = xt_ref[0]
    xrep = dot(xb, rmat)
    vals = (jax.lax.broadcasted_iota(jnp.int32, (M, BL * 10), 1) % 10).astype(f32)
    oh = (xrep == vals).astype(f32)
    z = jax.nn.relu(dot(oh, embk_ref[...]) + cellb)
    z = jax.nn.relu(dot(z, prek_ref[0]) + preb[0][None])
    z = jax.nn.relu(dot(z, prek_ref[1]) + preb[1][None])
    x0 = dot(z, prek_ref[2]) + preb[2][None]

    xpost = dot(x0, postxk_ref[...]) + postb0_ref[...]
    cur = x0
    hs = jnp.zeros((M, S), f32)
    cs = jnp.zeros((M, S), f32)

    for t in range(N_STEPS):
        curb = cur.astype(bf16)
        ab = jnp.dot(curb, w0k, preferred_element_type=f32)
        a = ab[:, :S].astype(bf16)
        b = ab[:, S:] + msgb0
        ae = jnp.dot(gs, a, preferred_element_type=f32)
        be = jnp.broadcast_to(b[:, None, :], (M, DEG, S)).reshape(EE, S)
        e = jax.nn.relu(ae + be)
        e = jax.nn.relu(dnf(e, msgk_ref[0]) + msgb[0][None])
        e = jax.nn.relu(dnf(e, msgk_ref[1]) + msgb[1][None])
        aggs = jnp.dot(st, e.astype(bf16), preferred_element_type=f32)

        z = jax.nn.relu(dnf(aggs, combk_ref[...]) + xpost)
        z = jax.nn.relu(dnf(z, postk_ref[0]) + postb[0][None])
        z = jax.nn.relu(dnf(z, postk_ref[1]) + postb[1][None])

        gates = dnf(z, wihk_ref[...]) + lstmb
        if t > 0:
            gates = gates + dnf(hs, whhk_ref[...])
        ig = jax.nn.sigmoid(gates[:, 0 * S:1 * S])
        gg = jnp.tanh(gates[:, 2 * S:3 * S])
        og = jax.nn.sigmoid(gates[:, 3 * S:4 * S])
        if t > 0:
            fg = jax.nn.sigmoid(gates[:, 1 * S:2 * S])
            cs = fg * cs + ig * gg
        else:
            cs = ig * gg
        hs = og * jnp.tanh(cs)
        cur = cs

    out_ref[0] = dot(cur, outk_ref[...]) + outb_ref[...]


def kernel(x, edge_index, rows_emb, cols_emb, init_emb, pre_W0, pre_b0,
           pre_Wr, pre_br, msg_W0, msg_b0, msg_Wr, msg_br, post_W0, post_b0,
           post_Wr, post_br, lstm_Wih, lstm_Whh, lstm_bih, lstm_bhh,
           out_W, out_b):
    f32 = jnp.float32
    bf16 = jnp.bfloat16
    batch = x.shape[0]
    G = batch // (BL * NSUB)
    eye = jnp.eye(BL, dtype=f32)

    def kron(w):
        k, f = w.shape
        return jnp.einsum('ab,kf->akbf', eye, w).reshape(BL * k, BL * f)

    def tileb(bvec):
        return jnp.tile(bvec, (BL,)).reshape(1, -1)

    e0 = edge_index[:E]
    order = jnp.argsort(e0[:, 1], stable=True)
    gs1 = jax.nn.one_hot(e0[order, 0], P, dtype=f32)
    gs = jnp.einsum('ab,ef->aebf', jnp.eye(NSUB, dtype=f32),
                    gs1).reshape(EE, M).astype(bf16)
    st = jnp.repeat(jnp.eye(M, dtype=bf16), DEG, axis=1)

    node = jnp.arange(P)
    cellb = (jnp.take(rows_emb, node // 9, axis=0) @ pre_W0[EMBED:2 * EMBED]
             + jnp.take(cols_emb, node % 9, axis=0) @ pre_W0[2 * EMBED:]
             + pre_b0)
    cellb_t = jnp.tile(cellb, (NSUB, BL))
    emb_proj = init_emb @ pre_W0[:EMBED]
    embk = jnp.einsum('ab,vf->avbf', eye, emb_proj).reshape(BL * 10, S)
    rmat = jnp.repeat(eye, 10, axis=1)

    prek = jnp.stack([kron(pre_Wr[i]) for i in range(3)])
    preb = jnp.stack([tileb(pre_br[i])[0] for i in range(3)])
    w0k = jnp.concatenate([kron(msg_W0[:LINEAR]), kron(msg_W0[LINEAR:])],
                          axis=1).astype(bf16)
    msgb0 = tileb(msg_b0)
    msgk = jnp.stack([kron(msg_Wr[i]) for i in range(2)]).astype(bf16)
    msgb = jnp.stack([tileb(msg_br[i])[0] for i in range(2)])

    combk = kron(msg_Wr[2] @ post_W0[:MSG]).astype(bf16)
    postxk = kron(post_W0[MSG:])
    postb0 = tileb(post_b0 + DEG * (msg_br[2] @ post_W0[:MSG]))
    postk = jnp.stack([kron(post_Wr[i]) for i in range(2)]).astype(bf16)
    postb = jnp.stack([tileb(post_br[i])[0] for i in range(2)])

    wih_f = post_Wr[2] @ lstm_Wih
    w4 = wih_f.reshape(LINEAR, 4, LSTM)
    wihk = (jnp.einsum('ab,kgf->akgbf', eye, w4)
            .reshape(S, 4 * S).astype(bf16))
    h4 = lstm_Whh.reshape(LSTM, 4, LSTM)
    whhk = (jnp.einsum('ab,kgf->akgbf', eye, h4)
            .reshape(S, 4 * S).astype(bf16))
    lb = lstm_bih + lstm_bhh + post_br[2] @ lstm_Wih
    lb4 = lb.reshape(4, 1, LSTM)
    lstmb = jnp.broadcast_to(lb4, (4, BL, LSTM)).reshape(1, 4 * S)

    outk = kron(out_W)
    outb = tileb(out_b)

    xt = (x.reshape(G, NSUB, BL, P).transpose(0, 1, 3, 2)
          .reshape(G, M, BL).astype(f32))

    full = lambda arr: pl.BlockSpec(arr.shape, lambda i: (0,) * arr.ndim)
    weights = [gs, st, rmat, embk, cellb_t, prek, preb, w0k, msgb0,
               msgk, msgb, combk, postxk, postb0, postk, postb,
               wihk, whhk, lstmb, outk, outb]
    in_specs = [pl.BlockSpec((1, M, BL), lambda i: (i, 0, 0))]
    in_specs += [full(w) for w in weights]

    out = pl.pallas_call(
        _rrn_block,
        grid=(G,),
        in_specs=in_specs,
        out_specs=pl.BlockSpec((1, M, BL * 9), lambda i: (i, 0, 0)),
        out_shape=jax.ShapeDtypeStruct((G, M, BL * 9), f32),
        compiler_params=pltpu.CompilerParams(
            dimension_semantics=("arbitrary",)),
    )(xt, *weights)

    return (out.reshape(G, NSUB, P, BL, 9).transpose(0, 1, 3, 2, 4)
            .reshape(batch, P, 9))
